# baseline (speedup 1.0000x reference)
import functools
import math

import jax
import jax.numpy as jnp
from jax import lax
from jax.experimental import pallas as pl
from jax.experimental.pallas import tpu as pltpu
from jax.experimental.pallas import tpu_sc as plsc

D_MODEL_ = 64
SCALE_ = math.sqrt(D_MODEL_)

_info = plsc.get_sparse_core_info()
_NC, _NS, _L = _info.num_cores, _info.num_subcores, _info.num_lanes
_NW = _NC * _NS

_CH = 128


def _make_sc_gather(N, V, D):
    assert N % (_NW * 2 * _CH) == 0
    ch_per_w = N // (_NW * _CH)
    n_pairs = ch_per_w // 2
    rows2 = _CH // 2
    mesh = plsc.VectorSubcoreMesh(core_axis_name="c", subcore_axis_name="s")

    @functools.partial(
        pl.kernel,
        mesh=mesh,
        out_type=jax.ShapeDtypeStruct((N, D), jnp.float32),
        scratch_types=[
            pltpu.VMEM((ch_per_w, _CH), jnp.int32),
            pltpu.VMEM((_CH, D), jnp.float32),
            pltpu.VMEM((_CH, D), jnp.float32),
            pltpu.VMEM((_CH, D), jnp.float32),
            pltpu.VMEM((_CH, D), jnp.float32),
            pltpu.SemaphoreType.DMA,
            pltpu.SemaphoreType.DMA,
            pltpu.SemaphoreType.DMA,
            pltpu.SemaphoreType.DMA,
        ],
        compiler_params=pltpu.CompilerParams(use_tc_tiling_on_sc=False),
    )
    def sc_gather(idx_hbm, table_hbm, out_hbm, idx_v, g0, g1, w0, w1,
                  gs0, gs1, ws0, ws1):
        wid = lax.axis_index("s") * _NC + lax.axis_index("c")
        pltpu.sync_copy(idx_hbm.at[pl.ds(wid * ch_per_w, ch_per_w)], idx_v)

        def gather(c, buf, sem):
            return pltpu.make_async_copy(
                table_hbm.at[idx_v.at[c]], buf, sem)

        def writeback(c, buf, sem):
            return pltpu.make_async_copy(
                buf,
                out_hbm.at[pl.ds((wid * ch_per_w + c) * _CH, _CH)], sem)

        def scale(src, dst):
            def row(i, carry):
                for j in range(D // _L):
                    sl = pl.ds(j * _L, _L)
                    dst[i, sl] = src[i, sl] * SCALE_
                return carry
            lax.fori_loop(0, _CH, row, 0, unroll=2)

        gather(0, g0, gs0).start()
        gather(1, g1, gs1).start()

        def pair(t, carry):
            c0 = 2 * t
            c1 = c0 + 1

            gather(c0, g0, gs0).wait()

            @pl.when(t > 0)
            def _():
                writeback(c0, w0, ws0).wait()

            scale(g0, w0)

            @pl.when(t < n_pairs - 1)
            def _():
                gather(c0 + 2, g0, gs0).start()

            writeback(c0, w0, ws0).start()

            gather(c1, g1, gs1).wait()

            @pl.when(t > 0)
            def _():
                writeback(c1, w1, ws1).wait()

            scale(g1, w1)

            @pl.when(t < n_pairs - 1)
            def _():
                gather(c1 + 2, g1, gs1).start()

            writeback(c1, w1, ws1).start()
            return carry

        lax.fori_loop(0, n_pairs, pair, 0)
        writeback(2 * n_pairs - 2, w0, ws0).wait()
        writeback(2 * n_pairs - 1, w1, ws1).wait()

    return sc_gather


def _tc_reformat_body(in_ref, out_ref, bs, S, D):
    out_ref[...] = in_ref[...].reshape(bs, S, D)


def _make_tc_reformat(B1, S, D, bs):
    assert B1 % bs == 0
    return pl.pallas_call(
        functools.partial(_tc_reformat_body, bs=bs, S=S, D=D),
        grid=(B1 // bs,),
        in_specs=[pl.BlockSpec((bs * S, D), lambda i: (i, 0))],
        out_specs=pl.BlockSpec((bs, S, D), lambda i: (i, 0, 0)),
        out_shape=jax.ShapeDtypeStruct((B1, S, D), jnp.float32),
    )


def kernel(x, table):
    B1, S = x.shape
    V, D = table.shape
    N = B1 * S
    idx2d = x.reshape(N // _CH, _CH).astype(jnp.int32)
    mid = _make_sc_gather(N, V, D)(idx2d, table)
    return _make_tc_reformat(B1, S, D, 256)(mid)


# speedup vs baseline: 1.2160x; 1.2160x over previous
import functools
import math

import jax
import jax.numpy as jnp
from jax import lax
from jax.experimental import pallas as pl
from jax.experimental.pallas import tpu as pltpu
from jax.experimental.pallas import tpu_sc as plsc

D_MODEL_ = 64
SCALE_ = math.sqrt(D_MODEL_)

_info = plsc.get_sparse_core_info()
_NC, _NS, _L = _info.num_cores, _info.num_subcores, _info.num_lanes
_NW = _NC * _NS

_CH = 128


def _make_sc_gather(N, V, D):
    assert N % (_NW * 2 * _CH) == 0
    ch_per_w = N // (_NW * _CH)
    n_pairs = ch_per_w // 2
    rows2 = _CH // 2
    mesh = plsc.VectorSubcoreMesh(core_axis_name="c", subcore_axis_name="s")

    @functools.partial(
        pl.kernel,
        mesh=mesh,
        out_type=jax.ShapeDtypeStruct((N // 2, 2 * D), jnp.float32),
        scratch_types=[
            pltpu.VMEM((ch_per_w, _CH), jnp.int32),
            pltpu.VMEM((_CH, D), jnp.float32),
            pltpu.VMEM((_CH, D), jnp.float32),
            pltpu.VMEM((rows2, 2 * D), jnp.float32),
            pltpu.VMEM((rows2, 2 * D), jnp.float32),
            pltpu.SemaphoreType.DMA,
            pltpu.SemaphoreType.DMA,
            pltpu.SemaphoreType.DMA,
            pltpu.SemaphoreType.DMA,
        ],
        compiler_params=pltpu.CompilerParams(use_tc_tiling_on_sc=False),
    )
    def sc_gather(idx_hbm, table_hbm, out_hbm, idx_v, g0, g1, w0, w1,
                  gs0, gs1, ws0, ws1):
        wid = lax.axis_index("s") * _NC + lax.axis_index("c")
        pltpu.sync_copy(idx_hbm.at[pl.ds(wid * ch_per_w, ch_per_w)], idx_v)

        def gather(c, buf, sem):
            return pltpu.make_async_copy(
                table_hbm.at[idx_v.at[c]], buf, sem)

        def writeback(c, buf, sem):
            return pltpu.make_async_copy(
                buf,
                out_hbm.at[pl.ds((wid * ch_per_w + c) * rows2, rows2)], sem)

        def scale(src, dst):
            def body(i, carry):
                for k in range(4):
                    q = i + 16 * k
                    for j in range(D // _L):
                        sl = pl.ds(j * _L, _L)
                        dst[q, pl.ds(j * _L, _L)] = (
                            src[2 * i + 32 * k, sl] * SCALE_)
                        dst[q, pl.ds(D + j * _L, _L)] = (
                            src[2 * i + 32 * k + 1, sl] * SCALE_)
                return carry
            lax.fori_loop(0, 16, body, 0)

        gather(0, g0, gs0).start()
        gather(1, g1, gs1).start()

        def pair(t, carry):
            c0 = 2 * t
            c1 = c0 + 1

            gather(c0, g0, gs0).wait()

            @pl.when(t > 0)
            def _():
                writeback(c0, w0, ws0).wait()

            scale(g0, w0)

            @pl.when(t < n_pairs - 1)
            def _():
                gather(c0 + 2, g0, gs0).start()

            writeback(c0, w0, ws0).start()

            gather(c1, g1, gs1).wait()

            @pl.when(t > 0)
            def _():
                writeback(c1, w1, ws1).wait()

            scale(g1, w1)

            @pl.when(t < n_pairs - 1)
            def _():
                gather(c1 + 2, g1, gs1).start()

            writeback(c1, w1, ws1).start()
            return carry

        lax.fori_loop(0, n_pairs, pair, 0)
        writeback(2 * n_pairs - 2, w0, ws0).wait()
        writeback(2 * n_pairs - 1, w1, ws1).wait()

    return sc_gather


def _tc_reformat_body(in_ref, out_ref, bs, S, D):
    b = in_ref[...].reshape(bs, S // 2, 2 * D)
    out_ref[:, ::2, :] = b[:, :, :D]
    out_ref[:, 1::2, :] = b[:, :, D:]


def _make_tc_reformat(B1, S, D, bs):
    assert B1 % bs == 0
    return pl.pallas_call(
        functools.partial(_tc_reformat_body, bs=bs, S=S, D=D),
        grid=(B1 // bs,),
        in_specs=[pl.BlockSpec((bs * S // 2, 2 * D), lambda i: (i, 0))],
        out_specs=pl.BlockSpec((bs, S, D), lambda i: (i, 0, 0)),
        out_shape=jax.ShapeDtypeStruct((B1, S, D), jnp.float32),
    )


def kernel(x, table):
    B1, S = x.shape
    V, D = table.shape
    N = B1 * S
    idx2d = x.reshape(N // _CH, _CH).astype(jnp.int32)
    mid = _make_sc_gather(N, V, D)(idx2d, table)
    return _make_tc_reformat(B1, S, D, 256)(mid)


# speedup vs baseline: 1.2252x; 1.0076x over previous
import functools
import math

import jax
import jax.numpy as jnp
from jax import lax
from jax.experimental import pallas as pl
from jax.experimental.pallas import tpu as pltpu
from jax.experimental.pallas import tpu_sc as plsc

D_MODEL_ = 64
SCALE_ = math.sqrt(D_MODEL_)

_info = plsc.get_sparse_core_info()
_NC, _NS, _L = _info.num_cores, _info.num_subcores, _info.num_lanes
_NW = _NC * _NS

_CH = 128


def _make_sc_gather(N, V, D):
    assert N % (_NW * 2 * _CH) == 0
    ch_per_w = N // (_NW * _CH)
    n_pairs = ch_per_w // 2
    rows2 = _CH // 2
    mesh = plsc.VectorSubcoreMesh(core_axis_name="c", subcore_axis_name="s")

    @functools.partial(
        pl.kernel,
        mesh=mesh,
        out_type=jax.ShapeDtypeStruct((N // 2, 2 * D), jnp.float32),
        scratch_types=[
            pltpu.VMEM((ch_per_w, _CH), jnp.int32),
            pltpu.VMEM((_CH, D), jnp.float32),
            pltpu.VMEM((_CH, D), jnp.float32),
            pltpu.VMEM((rows2, 2 * D), jnp.float32),
            pltpu.VMEM((rows2, 2 * D), jnp.float32),
            pltpu.SemaphoreType.DMA,
            pltpu.SemaphoreType.DMA,
            pltpu.SemaphoreType.DMA,
            pltpu.SemaphoreType.DMA,
        ],
        compiler_params=pltpu.CompilerParams(use_tc_tiling_on_sc=False),
    )
    def sc_gather(idx_hbm, table_hbm, out_hbm, idx_v, g0, g1, w0, w1,
                  gs0, gs1, ws0, ws1):
        wid = lax.axis_index("s") * _NC + lax.axis_index("c")
        pltpu.sync_copy(idx_hbm.at[pl.ds(wid * ch_per_w, ch_per_w)], idx_v)

        def gather(c, buf, sem):
            return pltpu.make_async_copy(
                table_hbm.at[idx_v.at[c]], buf, sem)

        def writeback(c, buf, sem):
            return pltpu.make_async_copy(
                buf,
                out_hbm.at[pl.ds((wid * ch_per_w + c) * rows2, rows2)], sem)

        def scale(src, dst):
            def body(i, carry):
                for k in range(4):
                    q = i + 16 * k
                    for j in range(D // _L):
                        sl = pl.ds(j * _L, _L)
                        dst[q, pl.ds(j * _L, _L)] = (
                            src[2 * i + 32 * k, sl] * SCALE_)
                        dst[q, pl.ds(D + j * _L, _L)] = (
                            src[2 * i + 32 * k + 1, sl] * SCALE_)
                return carry
            lax.fori_loop(0, 16, body, 0)

        gather(0, g0, gs0).start()
        gather(1, g1, gs1).start()

        def pair(t, carry):
            c0 = 2 * t
            c1 = c0 + 1

            gather(c0, g0, gs0).wait()

            @pl.when(t > 0)
            def _():
                writeback(c0, w0, ws0).wait()

            scale(g0, w0)

            @pl.when(t < n_pairs - 1)
            def _():
                gather(c0 + 2, g0, gs0).start()

            writeback(c0, w0, ws0).start()

            gather(c1, g1, gs1).wait()

            @pl.when(t > 0)
            def _():
                writeback(c1, w1, ws1).wait()

            scale(g1, w1)

            @pl.when(t < n_pairs - 1)
            def _():
                gather(c1 + 2, g1, gs1).start()

            writeback(c1, w1, ws1).start()
            return carry

        lax.fori_loop(0, n_pairs, pair, 0)
        writeback(2 * n_pairs - 2, w0, ws0).wait()
        writeback(2 * n_pairs - 1, w1, ws1).wait()

    return sc_gather


def _tc_reformat_body(in_ref, out_ref, bs, S, D):
    b = in_ref[...].reshape(bs, S // 2, 2 * D)
    out_ref[:, ::2, :] = b[:, :, :D]
    out_ref[:, 1::2, :] = b[:, :, D:]


def _make_tc_reformat(B1, S, D, bs):
    assert B1 % bs == 0
    return pl.pallas_call(
        functools.partial(_tc_reformat_body, bs=bs, S=S, D=D),
        grid=(B1 // bs,),
        in_specs=[pl.BlockSpec((bs * S // 2, 2 * D), lambda i: (i, 0))],
        out_specs=pl.BlockSpec((bs, S, D), lambda i: (i, 0, 0)),
        out_shape=jax.ShapeDtypeStruct((B1, S, D), jnp.float32),
    )


def kernel(x, table):
    B1, S = x.shape
    V, D = table.shape
    N = B1 * S
    idx2d = x.reshape(N // _CH, _CH).astype(jnp.int32)
    mid = _make_sc_gather(N, V, D)(idx2d, table)
    return mid.reshape(B1, S, D)


# speedup vs baseline: 1.2257x; 1.0004x over previous
import functools
import math

import jax
import jax.numpy as jnp
from jax import lax
from jax.experimental import pallas as pl
from jax.experimental.pallas import tpu as pltpu
from jax.experimental.pallas import tpu_sc as plsc

D_MODEL_ = 64
SCALE_ = math.sqrt(D_MODEL_)

_info = plsc.get_sparse_core_info()
_NC, _NS, _L = _info.num_cores, _info.num_subcores, _info.num_lanes
_NW = _NC * _NS

_CH = 128


def _make_sc_gather(N, V, D):
    assert N % (_NW * 2 * _CH) == 0
    ch_per_w = N // (_NW * _CH)
    n_pairs = ch_per_w // 2
    rows2 = _CH // 2
    mesh = plsc.VectorSubcoreMesh(core_axis_name="c", subcore_axis_name="s")

    @functools.partial(
        pl.kernel,
        mesh=mesh,
        out_type=jax.ShapeDtypeStruct((N // 2, 2 * D), jnp.float32),
        scratch_types=[
            pltpu.VMEM((ch_per_w, _CH), jnp.int32),
            pltpu.VMEM((_CH, D), jnp.float32),
            pltpu.VMEM((_CH, D), jnp.float32),
            pltpu.VMEM((rows2, 2 * D), jnp.float32),
            pltpu.VMEM((rows2, 2 * D), jnp.float32),
            pltpu.SemaphoreType.DMA,
            pltpu.SemaphoreType.DMA,
            pltpu.SemaphoreType.DMA,
            pltpu.SemaphoreType.DMA,
        ],
        compiler_params=pltpu.CompilerParams(use_tc_tiling_on_sc=False),
    )
    def sc_gather(idx_hbm, table_hbm, out_hbm, idx_v, g0, g1, w0, w1,
                  gs0, gs1, ws0, ws1):
        wid = lax.axis_index("s") * _NC + lax.axis_index("c")
        pltpu.sync_copy(idx_hbm.at[pl.ds(wid * ch_per_w, ch_per_w)], idx_v)

        def gather(c, buf, sem):
            return pltpu.make_async_copy(
                table_hbm.at[idx_v.at[c]], buf, sem)

        def writeback(c, buf, sem):
            return pltpu.make_async_copy(
                buf,
                out_hbm.at[pl.ds((wid * ch_per_w + c) * rows2, rows2)], sem)

        def scale(src, dst):
            def body(i, carry):
                for k in range(4):
                    q = i + 16 * k
                    for j in range(D // _L):
                        sl = pl.ds(j * _L, _L)
                        dst[q, pl.ds(j * _L, _L)] = (
                            src[2 * i + 32 * k, sl] * SCALE_)
                        dst[q, pl.ds(D + j * _L, _L)] = (
                            src[2 * i + 32 * k + 1, sl] * SCALE_)
                return carry
            lax.fori_loop(0, 16, body, 0)

        gather(0, g0, gs0).start()
        gather(1, g1, gs1).start()

        def pair(t, carry):
            c0 = 2 * t
            c1 = c0 + 1

            gather(c0, g0, gs0).wait()

            @pl.when(t > 0)
            def _():
                writeback(c0, w0, ws0).wait()

            scale(g0, w0)

            @pl.when(t < n_pairs - 1)
            def _():
                gather(c0 + 2, g0, gs0).start()

            writeback(c0, w0, ws0).start()

            gather(c1, g1, gs1).wait()

            @pl.when(t > 0)
            def _():
                writeback(c1, w1, ws1).wait()

            scale(g1, w1)

            @pl.when(t < n_pairs - 1)
            def _():
                gather(c1 + 2, g1, gs1).start()

            writeback(c1, w1, ws1).start()
            return carry

        lax.fori_loop(0, n_pairs, pair, 0)
        writeback(2 * n_pairs - 2, w0, ws0).wait()
        writeback(2 * n_pairs - 1, w1, ws1).wait()

    return sc_gather


def kernel(x, table):
    B1, S = x.shape
    V, D = table.shape
    N = B1 * S
    idx2d = x.reshape(N // _CH, _CH).astype(jnp.int32)
    mid = _make_sc_gather(N, V, D)(idx2d, table)
    return mid.reshape(B1, S, D)
